# baseline (speedup 1.0000x reference)
import functools

import jax
import jax.numpy as jnp
from jax import lax
from jax.experimental import pallas as pl
from jax.experimental.pallas import tpu as pltpu
from jax.experimental.pallas import tpu_sc as plsc

B = 16384
EMB = 128
NC = 2
NS = 16
NW = NC * NS
BPW = B // NW
CH = 64
NCH = BPW // CH
GRP = CH // 16


def _mf_body(u_id, i_id, user_emb, user_bias, item_emb, item_bias, mean, out,
             uidx, iidx, urows0, irows0, urows1, irows1, bu, bi, mv, mat, outv,
             sem0, sem1, sem_b, sem_o):
    c = lax.axis_index("c")
    s = lax.axis_index("s")
    wid = s * NC + c
    base = pl.multiple_of(wid * BPW, BPW)

    cp_ui = pltpu.async_copy(u_id.at[pl.ds(base, BPW)], uidx, sem_b)
    cp_ii = pltpu.async_copy(i_id.at[pl.ds(base, BPW)], iidx, sem_b)
    cp_mv = pltpu.async_copy(mean, mv.at[pl.ds(0, 1)], sem_b)
    cp_ui.wait()
    cp_ii.wait()
    cp_mv.wait()

    cp_bu = pltpu.async_copy(user_bias.at[uidx], bu, sem_b)
    cp_bi = pltpu.async_copy(item_bias.at[iidx], bi, sem_b)

    lane = lax.iota(jnp.int32, 16)
    mean_s = mv[pl.ds(0, 16)][0]

    bufs = [(urows0, irows0, sem0), (urows1, irows1, sem1)]

    def start(ci):
        ub, ib, sem = bufs[ci % 2]
        coff = ci * CH
        cu = pltpu.async_copy(user_emb.at[uidx.at[pl.ds(coff, CH)]], ub, sem)
        cv = pltpu.async_copy(item_emb.at[iidx.at[pl.ds(coff, CH)]], ib, sem)
        return cu, cv

    pend = start(0)
    cp_bu.wait()
    cp_bi.wait()

    out_cps = []
    for ci in range(NCH):
        nxt = start(ci + 1) if ci + 1 < NCH else None
        pend[0].wait()
        pend[1].wait()
        ub, ib, _ = bufs[ci % 2]
        coff = ci * CH

        def group(g, _, ub=ub, ib=ib, coff=coff):
            row0 = pl.multiple_of(g * 16, 16)
            for j in range(16):
                r = row0 + j
                acc = ub[r, pl.ds(0, 16)] * ib[r, pl.ds(0, 16)]
                for v in range(1, 8):
                    acc = acc + ub[r, pl.ds(16 * v, 16)] * ib[r, pl.ds(16 * v, 16)]
                plsc.store_scatter(mat, [lane, jnp.full((16,), j, jnp.int32)], acc)
            tot = mat[0, pl.ds(0, 16)]
            for l in range(1, 16):
                tot = tot + mat[l, pl.ds(0, 16)]
            off = pl.multiple_of(coff + row0, 16)
            outv[pl.ds(off, 16)] = (
                tot + bu[pl.ds(off, 16)] + bi[pl.ds(off, 16)] + mean_s)
            return 0

        lax.fori_loop(0, GRP, group, 0)
        out_cps.append(pltpu.async_copy(
            outv.at[pl.ds(coff, CH)], out.at[pl.ds(base + coff, CH)], sem_o))
        pend = nxt

    for cp in out_cps:
        cp.wait()


@functools.partial(jax.jit, donate_argnums=())
def _mf(u_id, i_id, user_emb, user_bias, item_emb, item_bias, mean):
    mesh = plsc.VectorSubcoreMesh(core_axis_name="c", subcore_axis_name="s")
    k = pl.kernel(
        _mf_body,
        mesh=mesh,
        compiler_params=pltpu.CompilerParams(needs_layout_passes=False),
        out_type=jax.ShapeDtypeStruct((B,), jnp.float32),
        scratch_types=[
            pltpu.VMEM((BPW,), jnp.int32),
            pltpu.VMEM((BPW,), jnp.int32),
            pltpu.VMEM((CH, EMB), jnp.float32),
            pltpu.VMEM((CH, EMB), jnp.float32),
            pltpu.VMEM((CH, EMB), jnp.float32),
            pltpu.VMEM((CH, EMB), jnp.float32),
            pltpu.VMEM((BPW,), jnp.float32),
            pltpu.VMEM((BPW,), jnp.float32),
            pltpu.VMEM((16,), jnp.float32),
            pltpu.VMEM((16, 17), jnp.float32),
            pltpu.VMEM((BPW,), jnp.float32),
            pltpu.SemaphoreType.DMA,
            pltpu.SemaphoreType.DMA,
            pltpu.SemaphoreType.DMA,
            pltpu.SemaphoreType.DMA,
        ],
    )
    return k(u_id, i_id, user_emb, user_bias, item_emb, item_bias, mean)


def kernel(u_id, i_id, user_emb, user_bias, item_emb, item_bias, mean):
    return _mf(u_id, i_id, user_emb, user_bias.reshape(-1), item_emb,
               item_bias.reshape(-1), mean)


# speedup vs baseline: 1.0206x; 1.0206x over previous
import functools

import jax
import jax.numpy as jnp
from jax import lax
from jax.experimental import pallas as pl
from jax.experimental.pallas import tpu as pltpu
from jax.experimental.pallas import tpu_sc as plsc

B = 16384
EMB = 128
NC = 2
NS = 16
NW = NC * NS
BPW = B // NW
CH = 128
NCH = BPW // CH
GRP = CH // 16


def _mf_body(u_id, i_id, user_emb, user_bias, item_emb, item_bias, mean, out,
             uidx, iidx, urows0, irows0, urows1, irows1, bu, bi, mv, mat, outv,
             sem0, sem1, sem_b, sem_o):
    c = lax.axis_index("c")
    s = lax.axis_index("s")
    wid = s * NC + c
    base = pl.multiple_of(wid * BPW, BPW)

    cp_ui = pltpu.async_copy(u_id.at[pl.ds(base, BPW)], uidx, sem_b)
    cp_ii = pltpu.async_copy(i_id.at[pl.ds(base, BPW)], iidx, sem_b)
    cp_mv = pltpu.async_copy(mean, mv.at[pl.ds(0, 1)], sem_b)
    cp_ui.wait()
    cp_ii.wait()
    cp_mv.wait()

    cp_bu = pltpu.async_copy(user_bias.at[uidx], bu, sem_b)
    cp_bi = pltpu.async_copy(item_bias.at[iidx], bi, sem_b)

    lane = lax.iota(jnp.int32, 16)
    mean_s = mv[pl.ds(0, 16)][0]

    bufs = [(urows0, irows0, sem0), (urows1, irows1, sem1)]

    def start(ci):
        ub, ib, sem = bufs[ci % 2]
        coff = ci * CH
        cu = pltpu.async_copy(user_emb.at[uidx.at[pl.ds(coff, CH)]], ub, sem)
        cv = pltpu.async_copy(item_emb.at[iidx.at[pl.ds(coff, CH)]], ib, sem)
        return cu, cv

    pend = start(0)
    cp_bu.wait()
    cp_bi.wait()

    out_cps = []
    for ci in range(NCH):
        nxt = start(ci + 1) if ci + 1 < NCH else None
        pend[0].wait()
        pend[1].wait()
        ub, ib, _ = bufs[ci % 2]
        coff = ci * CH

        def group(g, _, ub=ub, ib=ib, coff=coff):
            row0 = pl.multiple_of(g * 16, 16)
            for j in range(16):
                r = row0 + j
                acc = ub[r, pl.ds(0, 16)] * ib[r, pl.ds(0, 16)]
                for v in range(1, 8):
                    acc = acc + ub[r, pl.ds(16 * v, 16)] * ib[r, pl.ds(16 * v, 16)]
                plsc.store_scatter(mat, [lane, jnp.full((16,), j, jnp.int32)], acc)
            tot = mat[0, pl.ds(0, 16)]
            for l in range(1, 16):
                tot = tot + mat[l, pl.ds(0, 16)]
            off = pl.multiple_of(coff + row0, 16)
            outv[pl.ds(off, 16)] = (
                tot + bu[pl.ds(off, 16)] + bi[pl.ds(off, 16)] + mean_s)
            return 0

        lax.fori_loop(0, GRP, group, 0)
        out_cps.append(pltpu.async_copy(
            outv.at[pl.ds(coff, CH)], out.at[pl.ds(base + coff, CH)], sem_o))
        pend = nxt

    for cp in out_cps:
        cp.wait()


@functools.partial(jax.jit, donate_argnums=())
def _mf(u_id, i_id, user_emb, user_bias, item_emb, item_bias, mean):
    mesh = plsc.VectorSubcoreMesh(core_axis_name="c", subcore_axis_name="s")
    k = pl.kernel(
        _mf_body,
        mesh=mesh,
        compiler_params=pltpu.CompilerParams(needs_layout_passes=False),
        out_type=jax.ShapeDtypeStruct((B,), jnp.float32),
        scratch_types=[
            pltpu.VMEM((BPW,), jnp.int32),
            pltpu.VMEM((BPW,), jnp.int32),
            pltpu.VMEM((CH, EMB), jnp.float32),
            pltpu.VMEM((CH, EMB), jnp.float32),
            pltpu.VMEM((CH, EMB), jnp.float32),
            pltpu.VMEM((CH, EMB), jnp.float32),
            pltpu.VMEM((BPW,), jnp.float32),
            pltpu.VMEM((BPW,), jnp.float32),
            pltpu.VMEM((16,), jnp.float32),
            pltpu.VMEM((16, 17), jnp.float32),
            pltpu.VMEM((BPW,), jnp.float32),
            pltpu.SemaphoreType.DMA,
            pltpu.SemaphoreType.DMA,
            pltpu.SemaphoreType.DMA,
            pltpu.SemaphoreType.DMA,
        ],
    )
    return k(u_id, i_id, user_emb, user_bias, item_emb, item_bias, mean)


def kernel(u_id, i_id, user_emb, user_bias, item_emb, item_bias, mean):
    return _mf(u_id, i_id, user_emb, user_bias.reshape(-1), item_emb,
               item_bias.reshape(-1), mean)


# speedup vs baseline: 1.0493x; 1.0281x over previous
import functools

import jax
import jax.numpy as jnp
from jax import lax
from jax.experimental import pallas as pl
from jax.experimental.pallas import tpu as pltpu
from jax.experimental.pallas import tpu_sc as plsc

B = 16384
EMB = 128
NC = 2
NS = 16
NW = NC * NS
BPW = B // NW
CH = 128
NCH = BPW // CH
GRP = CH // 16


def _mf_body(u_id, i_id, user_emb, user_bias, item_emb, item_bias, mean, out,
             uidx, iidx, urows0, irows0, urows1, irows1, bu, bi, mv, mat, outv,
             sem0, sem1, sem_b, sem_o):
    c = lax.axis_index("c")
    s = lax.axis_index("s")
    wid = s * NC + c
    base = pl.multiple_of(wid * BPW, BPW)

    cp_ui = pltpu.async_copy(u_id.at[pl.ds(base, BPW)], uidx, sem_b)
    cp_ii = pltpu.async_copy(i_id.at[pl.ds(base, BPW)], iidx, sem_b)
    cp_mv = pltpu.async_copy(mean, mv.at[pl.ds(0, 1)], sem_b)
    cp_ui.wait()
    cp_ii.wait()
    cp_mv.wait()

    cp_bu = pltpu.async_copy(user_bias.at[uidx], bu, sem_b)
    cp_bi = pltpu.async_copy(item_bias.at[iidx], bi, sem_b)

    lane = lax.iota(jnp.int32, 16)
    mean_s = mv[pl.ds(0, 16)][0]

    bufs = [(urows0, irows0, sem0), (urows1, irows1, sem1)]

    def start(ci):
        ub, ib, sem = bufs[ci % 2]
        coff = ci * CH
        cu = pltpu.async_copy(user_emb.at[uidx.at[pl.ds(coff, CH)]], ub, sem)
        cv = pltpu.async_copy(item_emb.at[iidx.at[pl.ds(coff, CH)]], ib, sem)
        return cu, cv

    pend = start(0)
    cp_bu.wait()
    cp_bi.wait()

    out_cps = []
    for ci in range(NCH):
        nxt = start(ci + 1) if ci + 1 < NCH else None
        pend[0].wait()
        pend[1].wait()
        ub, ib, _ = bufs[ci % 2]
        coff = ci * CH

        def group(g, _, ub=ub, ib=ib, coff=coff):
            row0 = pl.multiple_of(g * 16, 16)
            def load_row(r):
                return ([ub[r, pl.ds(16 * v, 16)] for v in range(8)],
                        [ib[r, pl.ds(16 * v, 16)] for v in range(8)])

            cur = load_row(row0)
            for j in range(16):
                nxt = load_row(row0 + j + 1) if j < 15 else None
                us, vs = cur
                p = [us[v] * vs[v] for v in range(8)]
                while len(p) > 1:
                    p = [p[i] + p[i + 1] for i in range(0, len(p), 2)]
                plsc.store_scatter(mat, [lane, jnp.full((16,), j, jnp.int32)], p[0])
                cur = nxt
            cols = [mat[l, pl.ds(0, 16)] for l in range(16)]
            while len(cols) > 1:
                cols = [cols[i] + cols[i + 1] for i in range(0, len(cols), 2)]
            tot = cols[0]
            off = pl.multiple_of(coff + row0, 16)
            outv[pl.ds(off, 16)] = (
                tot + bu[pl.ds(off, 16)] + bi[pl.ds(off, 16)] + mean_s)
            return 0

        lax.fori_loop(0, GRP, group, 0)
        out_cps.append(pltpu.async_copy(
            outv.at[pl.ds(coff, CH)], out.at[pl.ds(base + coff, CH)], sem_o))
        pend = nxt

    for cp in out_cps:
        cp.wait()


@functools.partial(jax.jit, donate_argnums=())
def _mf(u_id, i_id, user_emb, user_bias, item_emb, item_bias, mean):
    mesh = plsc.VectorSubcoreMesh(core_axis_name="c", subcore_axis_name="s")
    k = pl.kernel(
        _mf_body,
        mesh=mesh,
        compiler_params=pltpu.CompilerParams(needs_layout_passes=False),
        out_type=jax.ShapeDtypeStruct((B,), jnp.float32),
        scratch_types=[
            pltpu.VMEM((BPW,), jnp.int32),
            pltpu.VMEM((BPW,), jnp.int32),
            pltpu.VMEM((CH, EMB), jnp.float32),
            pltpu.VMEM((CH, EMB), jnp.float32),
            pltpu.VMEM((CH, EMB), jnp.float32),
            pltpu.VMEM((CH, EMB), jnp.float32),
            pltpu.VMEM((BPW,), jnp.float32),
            pltpu.VMEM((BPW,), jnp.float32),
            pltpu.VMEM((16,), jnp.float32),
            pltpu.VMEM((16, 17), jnp.float32),
            pltpu.VMEM((BPW,), jnp.float32),
            pltpu.SemaphoreType.DMA,
            pltpu.SemaphoreType.DMA,
            pltpu.SemaphoreType.DMA,
            pltpu.SemaphoreType.DMA,
        ],
    )
    return k(u_id, i_id, user_emb, user_bias, item_emb, item_bias, mean)


def kernel(u_id, i_id, user_emb, user_bias, item_emb, item_bias, mean):
    return _mf(u_id, i_id, user_emb, user_bias.reshape(-1), item_emb,
               item_bias.reshape(-1), mean)
